# baseline (speedup 1.0000x reference)
import functools

import jax
import jax.numpy as jnp
from jax import lax
from jax.experimental import pallas as pl
from jax.experimental.pallas import tpu as pltpu
from jax.experimental.pallas import tpu_sc as plsc

N = 10000
E = 320000
D = 128
NPAD = 10240
NW = 32
RPW = 80
NROWS = NW * RPW
EPAD = NROWS * 128
ZROWS = NPAD // 16
BLK = 512
GRID = NPAD // BLK

_mesh = plsc.VectorSubcoreMesh(core_axis_name="c", subcore_axis_name="s")


HALF = RPW // 2


@functools.partial(
    pl.kernel,
    out_type=jax.ShapeDtypeStruct((2, NPAD, D), jnp.float32),
    mesh=_mesh,
    scratch_types=[
        pltpu.VMEM((HALF, 128), jnp.int32),
        pltpu.VMEM((HALF, 128), jnp.int32),
        [pltpu.VMEM((128, D), jnp.float32) for _ in range(2)],
        pltpu.VMEM_SHARED((NPAD, D), jnp.float32),
        pltpu.SemaphoreType.DMA,
        pltpu.SemaphoreType.DMA,
    ],
)
def _segsum(h_hbm, src_hbm, dst_hbm, zeros_hbm, out_hbm,
            src_v, dst_v, rows_v, acc_sh, sem_g, sem_s):
    c = lax.axis_index("c")
    s = lax.axis_index("s")
    w = (1 - c) * 16 + s
    pltpu.sync_copy(zeros_hbm, acc_sh.at[pl.ds(s * ZROWS, ZROWS)])
    plsc.subcore_barrier()

    NQ = 4

    def gath(j, par, q):
        sl = pl.ds(q * 32, 32)
        return pltpu.make_async_copy(
            h_hbm.at[src_v.at[j, sl]], rows_v[par].at[sl], sem_g)

    def scat(j, par):
        return pltpu.make_async_copy(
            rows_v[par], acc_sh.at[dst_v.at[j]], sem_s)

    def stage(j, par):
        scat(j, par).start(add=True)

        @pl.when(j > 0)
        def _():
            scat(j - 1, 1 - par).wait()

        jn = lax.rem(j + 1, HALF)
        for q in range(NQ):
            gath(jn, 1 - par, q).start()
        for q in range(NQ):
            gath(jn, 1 - par, q).wait()

    def body(u, carry):
        stage(2 * u, 0)
        stage(2 * u + 1, 1)
        return carry

    for h in range(2):
        pltpu.sync_copy(src_hbm.at[pl.ds(w * RPW + h * HALF, HALF)], src_v)
        pltpu.sync_copy(dst_hbm.at[pl.ds(w * RPW + h * HALF, HALF)], dst_v)
        for q in range(NQ):
            gath(0, 0, q).start()
        for q in range(NQ):
            gath(0, 0, q).wait()
        lax.fori_loop(0, HALF // 2, body, 0)
        scat(HALF - 1, 1).wait()
    plsc.subcore_barrier()
    pltpu.sync_copy(acc_sh.at[pl.ds(s * ZROWS, ZROWS)],
                    out_hbm.at[c, pl.ds(s * ZROWS, ZROWS)])


@functools.partial(
    pl.kernel,
    out_type=jax.ShapeDtypeStruct((2, NPAD, D), jnp.float32),
    mesh=_mesh,
    scratch_types=[
        pltpu.VMEM((RPW, 128), jnp.int32),
        pltpu.VMEM((128, D), jnp.float32),
        pltpu.VMEM_SHARED((NPAD, D), jnp.float32),
        pltpu.SemaphoreType.DMA,
    ],
)
def _deg(dst_hbm, ones_hbm, zeros_hbm, out_hbm, dst_v, ones_v, acc_sh, sem):
    c = lax.axis_index("c")
    s = lax.axis_index("s")
    w = c * 16 + s
    pltpu.sync_copy(zeros_hbm, acc_sh.at[pl.ds(s * ZROWS, ZROWS)])
    pltpu.sync_copy(ones_hbm, ones_v)
    pltpu.sync_copy(dst_hbm.at[pl.ds(w * RPW, RPW)], dst_v)
    plsc.subcore_barrier()

    DEPTH = 4

    def scat(j):
        return pltpu.make_async_copy(ones_v, acc_sh.at[dst_v.at[j]], sem)

    for j0 in range(DEPTH):
        scat(j0).start(add=True)

    def body(j, carry):
        @pl.when(j + DEPTH < RPW)
        def _():
            scat(j + DEPTH).start(add=True)

        scat(j).wait()
        return carry

    lax.fori_loop(0, RPW, body, 0)
    plsc.subcore_barrier()
    pltpu.sync_copy(acc_sh.at[pl.ds(s * ZROWS, ZROWS)],
                    out_hbm.at[c, pl.ds(s * ZROWS, ZROWS)])


def _a_of(deg_blk, i):
    degc = (deg_blk[0] + deg_blk[1])[:, 0:1]
    rows = lax.broadcasted_iota(jnp.int32, (BLK, 1), 0) + i * BLK
    return jnp.where(rows < N, lax.rsqrt(jnp.maximum(degc, 1.0)), 0.0)


def _mm_in_body(x_ref, w_ref, b_ref, deg_ref, o_ref):
    a = _a_of(deg_ref[...], pl.program_id(0))
    h = jnp.dot(x_ref[...], w_ref[...], preferred_element_type=jnp.float32)
    o_ref[...] = (h + b_ref[...]) * a


_mm_in = pl.pallas_call(
    _mm_in_body,
    grid=(GRID,),
    in_specs=[
        pl.BlockSpec((BLK, D), lambda i: (i, 0)),
        pl.BlockSpec((D, D), lambda i: (0, 0)),
        pl.BlockSpec((1, D), lambda i: (0, 0)),
        pl.BlockSpec((2, BLK, D), lambda i: (0, i, 0)),
    ],
    out_specs=pl.BlockSpec((BLK, D), lambda i: (i, 0)),
    out_shape=jax.ShapeDtypeStruct((NPAD, D), jnp.float32),
)


def _mm_mid_body(p_ref, deg_ref, w_ref, b_ref, o_ref):
    a = _a_of(deg_ref[...], pl.program_id(0))
    p = p_ref[...]
    hin = jnp.maximum((p[0] + p[1]) * a, 0.0)
    h = jnp.dot(hin, w_ref[...], preferred_element_type=jnp.float32)
    o_ref[...] = (h + b_ref[...]) * a


_mm_mid = pl.pallas_call(
    _mm_mid_body,
    grid=(GRID,),
    in_specs=[
        pl.BlockSpec((2, BLK, D), lambda i: (0, i, 0)),
        pl.BlockSpec((2, BLK, D), lambda i: (0, i, 0)),
        pl.BlockSpec((D, D), lambda i: (0, 0)),
        pl.BlockSpec((1, D), lambda i: (0, 0)),
    ],
    out_specs=pl.BlockSpec((BLK, D), lambda i: (i, 0)),
    out_shape=jax.ShapeDtypeStruct((NPAD, D), jnp.float32),
)


def _final_body(p_ref, deg_ref, o_ref):
    a = _a_of(deg_ref[...], pl.program_id(0))
    p = p_ref[...]
    o_ref[...] = jnp.maximum((p[0] + p[1]) * a, 0.0)


_final = pl.pallas_call(
    _final_body,
    grid=(GRID,),
    in_specs=[
        pl.BlockSpec((2, BLK, D), lambda i: (0, i, 0)),
        pl.BlockSpec((2, BLK, D), lambda i: (0, i, 0)),
    ],
    out_specs=pl.BlockSpec((BLK, D), lambda i: (i, 0)),
    out_shape=jax.ShapeDtypeStruct((NPAD, D), jnp.float32),
)


def kernel(x, edge_index, W0, b0, W1, b1):
    src = edge_index[0].astype(jnp.int32)
    dst = edge_index[1].astype(jnp.int32)
    padi = jnp.full((EPAD - E,), N, jnp.int32)
    src2 = jnp.concatenate([src, padi]).reshape(NROWS, 128)
    dst2 = jnp.concatenate([dst, padi]).reshape(NROWS, 128)
    xp = jnp.zeros((NPAD, D), jnp.float32).at[:N].set(x)
    zeros = jnp.zeros((ZROWS, D), jnp.float32)
    ones = jnp.ones((128, D), jnp.float32)
    b0r = b0.reshape(1, D)
    b1r = b1.reshape(1, D)

    deg2 = _deg(dst2, ones, zeros)
    h0 = _mm_in(xp, W0, b0r, deg2)
    p0 = _segsum(h0, src2, dst2, zeros)
    h1 = _mm_mid(p0, deg2, W1, b1r)
    p1 = _segsum(h1, src2, dst2, zeros)
    out = _final(p1, deg2)
    return out[:N]


# speedup vs baseline: 1.1296x; 1.1296x over previous
import functools

import jax
import jax.numpy as jnp
from jax import lax
from jax.experimental import pallas as pl
from jax.experimental.pallas import tpu as pltpu
from jax.experimental.pallas import tpu_sc as plsc

N = 10000
E = 320000
D = 128
NPAD = 10240
NW = 32
RPW = 80
NROWS = NW * RPW
EPAD = NROWS * 128
ZROWS = NPAD // 16
BLK = 512
GRID = NPAD // BLK

_mesh = plsc.VectorSubcoreMesh(core_axis_name="c", subcore_axis_name="s")


HALF = RPW // 2
FAST_C = 0


@functools.partial(
    pl.kernel,
    out_type=jax.ShapeDtypeStruct((2, NPAD, D), jnp.float32),
    mesh=_mesh,
    scratch_types=[
        pltpu.VMEM((HALF, 128), jnp.int32),
        pltpu.VMEM((HALF, 128), jnp.int32),
        [pltpu.VMEM((128, D), jnp.float32) for _ in range(2)],
        pltpu.VMEM_SHARED((NPAD, D), jnp.float32),
        pltpu.SemaphoreType.DMA,
        pltpu.SemaphoreType.DMA,
    ],
)
def _segsum(h_hbm, src_hbm, dst_hbm, zeros_hbm, out_hbm,
            src_v, dst_v, rows_v, acc_sh, sem_g, sem_s):
    c = lax.axis_index("c")
    s = lax.axis_index("s")
    nwin = jnp.where(c == FAST_C, 3, 1)
    base = jnp.where(c == FAST_C, s * (3 * HALF), 16 * 3 * HALF + s * HALF)
    pltpu.sync_copy(zeros_hbm, acc_sh.at[pl.ds(s * ZROWS, ZROWS)])
    plsc.subcore_barrier()

    NQ = 4

    def gath(j, par, q):
        sl = pl.ds(q * 32, 32)
        return pltpu.make_async_copy(
            h_hbm.at[src_v.at[j, sl]], rows_v[par].at[sl], sem_g)

    def scat(j, par):
        return pltpu.make_async_copy(
            rows_v[par], acc_sh.at[dst_v.at[j]], sem_s)

    def stage(j, par):
        scat(j, par).start(add=True)

        @pl.when(j > 0)
        def _():
            scat(j - 1, 1 - par).wait()

        jn = lax.rem(j + 1, HALF)
        for q in range(NQ):
            gath(jn, 1 - par, q).start()
        for q in range(NQ):
            gath(jn, 1 - par, q).wait()

    def body(u, carry):
        stage(2 * u, 0)
        stage(2 * u + 1, 1)
        return carry

    for h in range(3):
        @pl.when(h < nwin)
        def _():
            pltpu.sync_copy(src_hbm.at[pl.ds(base + h * HALF, HALF)], src_v)
            pltpu.sync_copy(dst_hbm.at[pl.ds(base + h * HALF, HALF)], dst_v)
            for q in range(NQ):
                gath(0, 0, q).start()
            for q in range(NQ):
                gath(0, 0, q).wait()
            lax.fori_loop(0, HALF // 2, body, 0)
            scat(HALF - 1, 1).wait()
    plsc.subcore_barrier()
    pltpu.sync_copy(acc_sh.at[pl.ds(s * ZROWS, ZROWS)],
                    out_hbm.at[c, pl.ds(s * ZROWS, ZROWS)])


@functools.partial(
    pl.kernel,
    out_type=jax.ShapeDtypeStruct((2, NPAD, D), jnp.float32),
    mesh=_mesh,
    scratch_types=[
        pltpu.VMEM((RPW, 128), jnp.int32),
        pltpu.VMEM((128, D), jnp.float32),
        pltpu.VMEM_SHARED((NPAD, D), jnp.float32),
        pltpu.SemaphoreType.DMA,
    ],
)
def _deg(dst_hbm, ones_hbm, zeros_hbm, out_hbm, dst_v, ones_v, acc_sh, sem):
    c = lax.axis_index("c")
    s = lax.axis_index("s")
    w = c * 16 + s
    pltpu.sync_copy(zeros_hbm, acc_sh.at[pl.ds(s * ZROWS, ZROWS)])
    pltpu.sync_copy(ones_hbm, ones_v)
    pltpu.sync_copy(dst_hbm.at[pl.ds(w * RPW, RPW)], dst_v)
    plsc.subcore_barrier()

    DEPTH = 4

    def scat(j):
        return pltpu.make_async_copy(ones_v, acc_sh.at[dst_v.at[j]], sem)

    for j0 in range(DEPTH):
        scat(j0).start(add=True)

    def body(j, carry):
        @pl.when(j + DEPTH < RPW)
        def _():
            scat(j + DEPTH).start(add=True)

        scat(j).wait()
        return carry

    lax.fori_loop(0, RPW, body, 0)
    plsc.subcore_barrier()
    pltpu.sync_copy(acc_sh.at[pl.ds(s * ZROWS, ZROWS)],
                    out_hbm.at[c, pl.ds(s * ZROWS, ZROWS)])


def _a_of(deg_blk, i):
    degc = (deg_blk[0] + deg_blk[1])[:, 0:1]
    rows = lax.broadcasted_iota(jnp.int32, (BLK, 1), 0) + i * BLK
    return jnp.where(rows < N, lax.rsqrt(jnp.maximum(degc, 1.0)), 0.0)


def _mm_in_body(x_ref, w_ref, b_ref, deg_ref, o_ref):
    a = _a_of(deg_ref[...], pl.program_id(0))
    h = jnp.dot(x_ref[...], w_ref[...], preferred_element_type=jnp.float32)
    o_ref[...] = (h + b_ref[...]) * a


_mm_in = pl.pallas_call(
    _mm_in_body,
    grid=(GRID,),
    in_specs=[
        pl.BlockSpec((BLK, D), lambda i: (i, 0)),
        pl.BlockSpec((D, D), lambda i: (0, 0)),
        pl.BlockSpec((1, D), lambda i: (0, 0)),
        pl.BlockSpec((2, BLK, D), lambda i: (0, i, 0)),
    ],
    out_specs=pl.BlockSpec((BLK, D), lambda i: (i, 0)),
    out_shape=jax.ShapeDtypeStruct((NPAD, D), jnp.float32),
)


def _mm_mid_body(p_ref, deg_ref, w_ref, b_ref, o_ref):
    a = _a_of(deg_ref[...], pl.program_id(0))
    p = p_ref[...]
    hin = jnp.maximum((p[0] + p[1]) * a, 0.0)
    h = jnp.dot(hin, w_ref[...], preferred_element_type=jnp.float32)
    o_ref[...] = (h + b_ref[...]) * a


_mm_mid = pl.pallas_call(
    _mm_mid_body,
    grid=(GRID,),
    in_specs=[
        pl.BlockSpec((2, BLK, D), lambda i: (0, i, 0)),
        pl.BlockSpec((2, BLK, D), lambda i: (0, i, 0)),
        pl.BlockSpec((D, D), lambda i: (0, 0)),
        pl.BlockSpec((1, D), lambda i: (0, 0)),
    ],
    out_specs=pl.BlockSpec((BLK, D), lambda i: (i, 0)),
    out_shape=jax.ShapeDtypeStruct((NPAD, D), jnp.float32),
)


def _final_body(p_ref, deg_ref, o_ref):
    a = _a_of(deg_ref[...], pl.program_id(0))
    p = p_ref[...]
    o_ref[...] = jnp.maximum((p[0] + p[1]) * a, 0.0)


_final = pl.pallas_call(
    _final_body,
    grid=(GRID,),
    in_specs=[
        pl.BlockSpec((2, BLK, D), lambda i: (0, i, 0)),
        pl.BlockSpec((2, BLK, D), lambda i: (0, i, 0)),
    ],
    out_specs=pl.BlockSpec((BLK, D), lambda i: (i, 0)),
    out_shape=jax.ShapeDtypeStruct((NPAD, D), jnp.float32),
)


def kernel(x, edge_index, W0, b0, W1, b1):
    src = edge_index[0].astype(jnp.int32)
    dst = edge_index[1].astype(jnp.int32)
    padi = jnp.full((EPAD - E,), N, jnp.int32)
    src2 = jnp.concatenate([src, padi]).reshape(NROWS, 128)
    dst2 = jnp.concatenate([dst, padi]).reshape(NROWS, 128)
    xp = jnp.zeros((NPAD, D), jnp.float32).at[:N].set(x)
    zeros = jnp.zeros((ZROWS, D), jnp.float32)
    ones = jnp.ones((128, D), jnp.float32)
    b0r = b0.reshape(1, D)
    b1r = b1.reshape(1, D)

    deg2 = _deg(dst2, ones, zeros)
    h0 = _mm_in(xp, W0, b0r, deg2)
    p0 = _segsum(h0, src2, dst2, zeros)
    h1 = _mm_mid(p0, deg2, W1, b1r)
    p1 = _segsum(h1, src2, dst2, zeros)
    out = _final(p1, deg2)
    return out[:N]
